# Initial kernel scaffold; baseline (speedup 1.0000x reference)
import functools

import jax
import jax.numpy as jnp
from jax import lax
from jax.experimental import pallas as pl
from jax.experimental.pallas import tpu as pltpu
from jax.experimental.pallas import tpu_sc as plsc

N = 10000
E = 320000
D = 128

NC = 2
NS = 16
NW = NC * NS
CH = 128
NCHUNK = E // CH
NACC = N + 8
TRASH = N
ROWS_PER_SUB = N // NS


def _sc_aggregate(x, edge_index):
    mesh = plsc.VectorSubcoreMesh(
        core_axis_name="c", subcore_axis_name="s",
        num_cores=NC, num_subcores=NS,
    )

    @functools.partial(
        pl.kernel,
        mesh=mesh,
        out_type=jax.ShapeDtypeStruct((NC, N, D), jnp.float32),
        scratch_types=[
            pltpu.VMEM((CH,), jnp.int32),
            pltpu.VMEM((CH,), jnp.int32),
            pltpu.VMEM((CH,), jnp.int32),
            pltpu.VMEM((CH, D), jnp.float32),
            pltpu.VMEM_SHARED((NACC, D), jnp.float32),
            pltpu.SemaphoreType.DMA,
        ],
    )
    def sc_kernel(x_hbm, e_hbm, out_hbm, row_v, col_v, dst_v, gath_v,
                  acc_sh, sem):
        core = lax.axis_index("c")
        sid = lax.axis_index("s")
        wid = sid * NC + core

        base = sid * ROWS_PER_SUB
        pltpu.sync_copy(x_hbm.at[pl.ds(base, ROWS_PER_SUB)],
                        acc_sh.at[pl.ds(base, ROWS_PER_SUB)])
        plsc.subcore_barrier()

        nj = jnp.where(wid < NCHUNK % NW, NCHUNK // NW + 1, NCHUNK // NW)

        def body(j, carry):
            cid = wid + j * NW
            off = cid * CH
            pltpu.sync_copy(e_hbm.at[0, pl.ds(off, CH)], row_v)
            pltpu.sync_copy(e_hbm.at[1, pl.ds(off, CH)], col_v)
            for i in range(CH // 16):
                r = row_v[pl.ds(i * 16, 16)]
                c = col_v[pl.ds(i * 16, 16)]
                dst_v[pl.ds(i * 16, 16)] = jnp.where(r == c, TRASH, r)
            pltpu.async_copy(x_hbm.at[col_v], gath_v, sem).wait()
            pltpu.sync_copy(gath_v, acc_sh.at[dst_v], add=True)
            return carry

        lax.fori_loop(0, nj, body, 0)
        plsc.subcore_barrier()

        pltpu.sync_copy(acc_sh.at[pl.ds(base, ROWS_PER_SUB)],
                        out_hbm.at[core, pl.ds(base, ROWS_PER_SUB)])

    return sc_kernel(x, edge_index)


def _tc_finish(p0, p1, x, W):
    BR = 1000

    def body(p0_ref, p1_ref, x_ref, w_ref, o_ref):
        s = p0_ref[...] + p1_ref[...] - x_ref[...]
        y = jnp.dot(s, w_ref[...], preferred_element_type=jnp.float32)
        nrm = jnp.sqrt(jnp.sum(y * y, axis=1, keepdims=True))
        o_ref[...] = y / jnp.maximum(nrm, 1e-12)

    row_spec = pl.BlockSpec((BR, D), lambda i: (i, 0))
    return pl.pallas_call(
        body,
        grid=(N // BR,),
        in_specs=[row_spec, row_spec, row_spec,
                  pl.BlockSpec((D, D), lambda i: (0, 0))],
        out_specs=row_spec,
        out_shape=jax.ShapeDtypeStruct((N, D), jnp.float32),
    )(p0, p1, x, W)


@jax.jit
def kernel(x, edge_index, W):
    partials = _sc_aggregate(x, edge_index)
    return _tc_finish(partials[0], partials[1], x, W)


# speedup vs baseline: 10.8535x; 10.8535x over previous
import functools

import jax
import jax.numpy as jnp
from jax import lax
from jax.experimental import pallas as pl
from jax.experimental.pallas import tpu as pltpu
from jax.experimental.pallas import tpu_sc as plsc

N = 10000
E = 320000
D = 128

NC = 2
NS = 16
NW = NC * NS
CH = 128
NCHUNK = E // CH
NACC = N + 8
TRASH = N
PER = 624
TAIL = N - NS * PER


def _sc_aggregate(x, row, col):
    mesh = plsc.VectorSubcoreMesh(
        core_axis_name="c", subcore_axis_name="s",
        num_cores=NC, num_subcores=NS,
    )

    @functools.partial(
        pl.kernel,
        mesh=mesh,
        out_type=jax.ShapeDtypeStruct((NC, N, D), jnp.float32),
        scratch_types=[
            pltpu.VMEM((CH,), jnp.int32),
            pltpu.VMEM((CH,), jnp.int32),
            pltpu.VMEM((CH,), jnp.int32),
            pltpu.VMEM((CH, D), jnp.float32),
            pltpu.VMEM_SHARED((NACC, D), jnp.float32),
            pltpu.SemaphoreType.DMA,
        ],
    )
    def sc_kernel(x_hbm, row_hbm, col_hbm, out_hbm, row_v, col_v, dst_v,
                  gath_v, acc_sh, sem):
        core = lax.axis_index("c")
        sid = lax.axis_index("s")
        wid = sid * NC + core

        base = sid * PER
        pltpu.sync_copy(x_hbm.at[pl.ds(base, PER)],
                        acc_sh.at[pl.ds(base, PER)])

        @pl.when(sid == 0)
        def _():
            pltpu.sync_copy(x_hbm.at[pl.ds(NS * PER, TAIL)],
                            acc_sh.at[pl.ds(NS * PER, TAIL)])

        plsc.subcore_barrier()

        nj = jnp.where(wid < NCHUNK % NW, NCHUNK // NW + 1, NCHUNK // NW)

        def body(j, carry):
            cid = wid + j * NW
            off = cid * CH
            pltpu.sync_copy(row_hbm.at[pl.ds(off, CH)], row_v)
            pltpu.sync_copy(col_hbm.at[pl.ds(off, CH)], col_v)
            for i in range(CH // 16):
                r = row_v[pl.ds(i * 16, 16)]
                c = col_v[pl.ds(i * 16, 16)]
                dst_v[pl.ds(i * 16, 16)] = jnp.where(r == c, TRASH, r)
            pltpu.async_copy(x_hbm.at[col_v], gath_v, sem).wait()
            pltpu.sync_copy(gath_v, acc_sh.at[dst_v], add=True)
            return carry

        lax.fori_loop(0, nj, body, 0)
        plsc.subcore_barrier()

        pltpu.sync_copy(acc_sh.at[pl.ds(base, PER)],
                        out_hbm.at[core, pl.ds(base, PER)])

        @pl.when(sid == 0)
        def _():
            pltpu.sync_copy(acc_sh.at[pl.ds(NS * PER, TAIL)],
                            out_hbm.at[core, pl.ds(NS * PER, TAIL)])

    return sc_kernel(x, row, col)


def _tc_finish(p0, p1, x, W):
    BR = 1000

    def body(p0_ref, p1_ref, x_ref, w_ref, o_ref):
        s = p0_ref[...] + p1_ref[...] - x_ref[...]
        y = jnp.dot(s, w_ref[...], preferred_element_type=jnp.float32)
        nrm = jnp.sqrt(jnp.sum(y * y, axis=1, keepdims=True))
        o_ref[...] = y / jnp.maximum(nrm, 1e-12)

    row_spec = pl.BlockSpec((BR, D), lambda i: (i, 0))
    return pl.pallas_call(
        body,
        grid=(N // BR,),
        in_specs=[row_spec, row_spec, row_spec,
                  pl.BlockSpec((D, D), lambda i: (0, 0))],
        out_specs=row_spec,
        out_shape=jax.ShapeDtypeStruct((N, D), jnp.float32),
    )(p0, p1, x, W)


@jax.jit
def kernel(x, edge_index, W):
    partials = _sc_aggregate(x, edge_index[0], edge_index[1])
    return _tc_finish(partials[0], partials[1], x, W)
